# baseline (speedup 1.0000x reference)
import functools

import jax
import jax.numpy as jnp
from jax import lax
from jax.experimental import pallas as pl
from jax.experimental.pallas import tpu as pltpu
from jax.experimental.pallas import tpu_sc as plsc

NC = 2
NS = 16
NW = NC * NS
D = 32
BT = 128
HC = 4
RC = HC * BT


@functools.lru_cache(maxsize=None)
def _gather_call(H: int, B: int):
    assert B == NW * BT and H % (2 * HC) == 0
    nchunk = H // HC
    mesh = plsc.VectorSubcoreMesh(core_axis_name="c", subcore_axis_name="s")

    @functools.partial(
        pl.kernel,
        mesh=mesh,
        out_type=jax.ShapeDtypeStruct((H, D // 8, NW, 8, BT), jnp.float32),
        scratch_types=[
            pltpu.VMEM((nchunk, RC), jnp.int32),
            pltpu.VMEM((RC, D), jnp.float32),
            pltpu.VMEM((RC, D), jnp.float32),
            pltpu.VMEM((HC, D // 8, 8, BT), jnp.float32),
            pltpu.VMEM((HC, D // 8, 8, BT), jnp.float32),
            pltpu.SemaphoreType.DMA,
            pltpu.SemaphoreType.DMA,
        ],
        compiler_params=pltpu.CompilerParams(
            use_tc_tiling_on_sc=False, needs_layout_passes=False
        ),
    )
    def k(idx_hbm, table_hbm, out_hbm, idx_v, rows0, rows1, tr0, tr1, gsem, ssem):
        wid = lax.axis_index("s") * NC + lax.axis_index("c")
        pltpu.sync_copy(idx_hbm.at[wid], idx_v)
        rows = (rows0, rows1)
        trs = (tr0, tr1)

        lane = lax.iota(jnp.int32, 16)

        def gather(j, p):
            return pltpu.make_async_copy(
                table_hbm.at[idx_v.at[j]], rows[p], gsem
            )

        def store(j, p):
            return pltpu.make_async_copy(
                trs[p], out_hbm.at[pl.ds(j * HC, HC), :, wid], ssem
            )

        def transpose(p):
            def body(i, carry):
                vb = lane + i * 16
                hq = i // (BT // 16)
                il = i % (BT // 16)
                for c in range(D):
                    x = plsc.load_gather(
                        rows[p], [vb, jnp.full((16,), c, jnp.int32)]
                    )
                    trs[p][hq, c // 8, c % 8, pl.ds(il * 16, 16)] = x
                return carry

            lax.fori_loop(0, RC // 16, body, 0)

        gather(0, 0).start()
        gather(1, 1).start()
        for p in (0, 1):
            gather(p, p).wait()
            transpose(p)
            store(p, p).start()
            gather(p + 2, p).start()

        def body(i, carry):
            for p in (0, 1):
                j = i * 2 + p
                gather(j, p).wait()
                store(j - 2, p).wait()
                transpose(p)
                store(j, p).start()
                gather(j + 2, p).start()
            return carry

        lax.fori_loop(1, nchunk // 2 - 1, body, 0)

        for p in (0, 1):
            j = nchunk - 2 + p
            gather(j, p).wait()
            store(j - 2, p).wait()
            transpose(p)
            store(j, p).start()
        store(nchunk - 2, 0).wait()
        store(nchunk - 1, 1).wait()

    return k


def kernel(words_pretrained, table):
    b0, hist = words_pretrained.shape
    nchunk = hist // HC
    idx_arr = (
        words_pretrained.T.astype(jnp.int32)
        .reshape(hist, NW, BT)
        .transpose(1, 0, 2)
        .reshape(NW, nchunk, RC)
    )
    out5 = _gather_call(hist, b0)(idx_arr, table)
    return out5.transpose(2, 4, 0, 1, 3).reshape(b0, hist, D)


# speedup vs baseline: 1.2854x; 1.2854x over previous
import functools

import jax
import jax.numpy as jnp
from jax import lax
from jax.experimental import pallas as pl
from jax.experimental.pallas import tpu as pltpu
from jax.experimental.pallas import tpu_sc as plsc

NC = 2
NS = 16
NW = NC * NS
D = 32
BT = 128
HC = 4
RC = HC * BT


@functools.lru_cache(maxsize=None)
def _gather_call(H: int, B: int):
    assert B == NW * BT and H % (2 * HC) == 0
    nchunk = H // HC
    mesh = plsc.VectorSubcoreMesh(core_axis_name="c", subcore_axis_name="s")

    @functools.partial(
        pl.kernel,
        mesh=mesh,
        out_type=jax.ShapeDtypeStruct((H, D // 8, NW, 8, BT), jnp.float32),
        scratch_types=[
            pltpu.VMEM((nchunk, RC), jnp.int32),
            pltpu.VMEM((RC, D), jnp.float32),
            pltpu.VMEM((RC, D), jnp.float32),
            pltpu.VMEM((HC, D // 8, 8, BT), jnp.float32),
            pltpu.VMEM((HC, D // 8, 8, BT), jnp.float32),
            pltpu.SemaphoreType.DMA,
            pltpu.SemaphoreType.DMA,
        ],
        compiler_params=pltpu.CompilerParams(
            use_tc_tiling_on_sc=False, needs_layout_passes=False
        ),
    )
    def k(idx_hbm, table_hbm, out_hbm, idx_v, rows0, rows1, tr0, tr1, gsem, ssem):
        wid = lax.axis_index("s") * NC + lax.axis_index("c")
        pltpu.sync_copy(idx_hbm.at[wid], idx_v)
        rows = (rows0, rows1)
        trs = (tr0, tr1)

        lane = lax.iota(jnp.int32, 16)

        def gather(j, p):
            return pltpu.make_async_copy(
                table_hbm.at[idx_v.at[j]], rows[p], gsem
            )

        def store(j, p):
            return pltpu.make_async_copy(
                trs[p], out_hbm.at[pl.ds(j * HC, HC), :, wid], ssem
            )

        def transpose(p):
            def body(i, carry):
                vb = lane + i * 16
                hq = i // (BT // 16)
                il = i % (BT // 16)
                for c0 in range(0, D, 8):
                    xs = [
                        plsc.load_gather(
                            rows[p], [vb, jnp.full((16,), c0 + t, jnp.int32)]
                        )
                        for t in range(8)
                    ]
                    for t in range(8):
                        c = c0 + t
                        trs[p][hq, c // 8, c % 8, pl.ds(il * 16, 16)] = xs[t]
                return carry

            lax.fori_loop(0, RC // 16, body, 0)

        gather(0, 0).start()
        gather(1, 1).start()
        for p in (0, 1):
            gather(p, p).wait()
            transpose(p)
            store(p, p).start()
            gather(p + 2, p).start()

        def body(i, carry):
            for p in (0, 1):
                j = i * 2 + p
                gather(j, p).wait()
                store(j - 2, p).wait()
                transpose(p)
                store(j, p).start()
                gather(j + 2, p).start()
            return carry

        lax.fori_loop(1, nchunk // 2 - 1, body, 0)

        for p in (0, 1):
            j = nchunk - 2 + p
            gather(j, p).wait()
            store(j - 2, p).wait()
            transpose(p)
            store(j, p).start()
        store(nchunk - 2, 0).wait()
        store(nchunk - 1, 1).wait()

    return k


def kernel(words_pretrained, table):
    b0, hist = words_pretrained.shape
    nchunk = hist // HC
    idx_arr = (
        words_pretrained.T.astype(jnp.int32)
        .reshape(hist, NW, BT)
        .transpose(1, 0, 2)
        .reshape(NW, nchunk, RC)
    )
    out5 = _gather_call(hist, b0)(idx_arr, table)
    return out5.transpose(2, 4, 0, 1, 3).reshape(b0, hist, D)


# speedup vs baseline: 1.3102x; 1.0193x over previous
import functools

import jax
import jax.numpy as jnp
from jax import lax
from jax.experimental import pallas as pl
from jax.experimental.pallas import tpu as pltpu
from jax.experimental.pallas import tpu_sc as plsc

NC = 2
NS = 16
NW = NC * NS
D = 32
BT = 128
HC = 4
RC = HC * BT
VC = 2048


@functools.lru_cache(maxsize=None)
def _retile_call(vocab: int):
    grid = (vocab + VC - 1) // VC

    def body(t_ref, o_ref):
        x = t_ref[...]
        xt = x.T.reshape(VC // 4, 4, D)
        parts = [xt[:, q, :] for q in range(4)]
        o_ref[...] = jnp.concatenate(parts, axis=1)

    return pl.pallas_call(
        body,
        grid=(grid,),
        in_specs=[pl.BlockSpec((D, VC), lambda i: (0, i))],
        out_specs=pl.BlockSpec((VC // 4, 4 * D), lambda i: (i, 0)),
        out_shape=jax.ShapeDtypeStruct((vocab // 4, 4 * D), jnp.float32),
    )


@functools.lru_cache(maxsize=None)
def _gather_call(H: int, B: int):
    assert B == NW * BT and H % (2 * HC) == 0
    nchunk = H // HC
    mesh = plsc.VectorSubcoreMesh(core_axis_name="c", subcore_axis_name="s")

    @functools.partial(
        pl.kernel,
        mesh=mesh,
        out_type=jax.ShapeDtypeStruct((H, D // 8, NW, 8, BT), jnp.float32),
        scratch_types=[
            pltpu.VMEM((nchunk, RC), jnp.int32),
            pltpu.VMEM((RC, D), jnp.float32),
            pltpu.VMEM((RC, D), jnp.float32),
            pltpu.VMEM((HC, D // 8, 8, BT), jnp.float32),
            pltpu.VMEM((HC, D // 8, 8, BT), jnp.float32),
            pltpu.SemaphoreType.DMA,
            pltpu.SemaphoreType.DMA,
        ],
        compiler_params=pltpu.CompilerParams(
            use_tc_tiling_on_sc=False, needs_layout_passes=False
        ),
    )
    def k(idx_hbm, table_hbm, out_hbm, idx_v, rows0, rows1, tr0, tr1, gsem, ssem):
        wid = lax.axis_index("s") * NC + lax.axis_index("c")
        pltpu.sync_copy(idx_hbm.at[wid], idx_v)
        rows = (rows0, rows1)
        trs = (tr0, tr1)

        lane = lax.iota(jnp.int32, 16)

        def gather(j, p):
            return pltpu.make_async_copy(
                table_hbm.at[idx_v.at[j]], rows[p], gsem
            )

        def store(j, p):
            return pltpu.make_async_copy(
                trs[p], out_hbm.at[pl.ds(j * HC, HC), :, wid], ssem
            )

        def transpose(p):
            def body(i, carry):
                vb = lane + i * 16
                hq = i // (BT // 16)
                il = i % (BT // 16)
                for c0 in range(0, D, 8):
                    xs = [
                        plsc.load_gather(
                            rows[p], [vb, jnp.full((16,), c0 + t, jnp.int32)]
                        )
                        for t in range(8)
                    ]
                    for t in range(8):
                        c = c0 + t
                        trs[p][hq, c // 8, c % 8, pl.ds(il * 16, 16)] = xs[t]
                return carry

            lax.fori_loop(0, RC // 16, body, 0)

        gather(0, 0).start()
        gather(1, 1).start()
        for p in (0, 1):
            gather(p, p).wait()
            transpose(p)
            store(p, p).start()
            gather(p + 2, p).start()

        def body(i, carry):
            for p in (0, 1):
                j = i * 2 + p
                gather(j, p).wait()
                store(j - 2, p).wait()
                transpose(p)
                store(j, p).start()
                gather(j + 2, p).start()
            return carry

        lax.fori_loop(1, nchunk // 2 - 1, body, 0)

        for p in (0, 1):
            j = nchunk - 2 + p
            gather(j, p).wait()
            store(j - 2, p).wait()
            transpose(p)
            store(j, p).start()
        store(nchunk - 2, 0).wait()
        store(nchunk - 1, 1).wait()

    return k


def kernel(words_pretrained, table):
    b0, hist = words_pretrained.shape
    nchunk = hist // HC
    idx_arr = (
        words_pretrained.T.astype(jnp.int32)
        .reshape(hist, NW, BT)
        .transpose(1, 0, 2)
        .reshape(NW, nchunk, RC)
    )
    vocab = table.shape[0]
    tlin = _retile_call(vocab)(table.T).reshape(vocab, D)
    out5 = _gather_call(hist, b0)(idx_arr, tlin)
    return out5.transpose(2, 4, 0, 1, 3).reshape(b0, hist, D)


# speedup vs baseline: 1.3321x; 1.0168x over previous
import functools

import jax
import jax.numpy as jnp
from jax import lax
from jax.experimental import pallas as pl
from jax.experimental.pallas import tpu as pltpu
from jax.experimental.pallas import tpu_sc as plsc

NC = 2
NS = 16
NW = NC * NS
D = 32
BT = 128
HC = 5
RC = HC * BT
VC = 2048


@functools.lru_cache(maxsize=None)
def _retile_call(vocab: int):
    grid = (vocab + VC - 1) // VC

    def body(t_ref, o_ref):
        x = t_ref[...]
        xt = x.T.reshape(VC // 4, 4, D)
        parts = [xt[:, q, :] for q in range(4)]
        o_ref[...] = jnp.concatenate(parts, axis=1)

    return pl.pallas_call(
        body,
        grid=(grid,),
        in_specs=[pl.BlockSpec((D, VC), lambda i: (0, i))],
        out_specs=pl.BlockSpec((VC // 4, 4 * D), lambda i: (i, 0)),
        out_shape=jax.ShapeDtypeStruct((vocab // 4, 4 * D), jnp.float32),
    )


@functools.lru_cache(maxsize=None)
def _gather_call(H: int, B: int):
    assert B == NW * BT and H % (2 * HC) == 0
    nchunk = H // HC
    mesh = plsc.VectorSubcoreMesh(core_axis_name="c", subcore_axis_name="s")

    @functools.partial(
        pl.kernel,
        mesh=mesh,
        out_type=jax.ShapeDtypeStruct((H, D // 8, NW, 8, BT), jnp.float32),
        scratch_types=[
            pltpu.VMEM((nchunk, RC), jnp.int32),
            pltpu.VMEM((RC, D), jnp.float32),
            pltpu.VMEM((RC, D), jnp.float32),
            pltpu.VMEM((HC, D // 8, 8, BT), jnp.float32),
            pltpu.VMEM((HC, D // 8, 8, BT), jnp.float32),
            pltpu.SemaphoreType.DMA,
            pltpu.SemaphoreType.DMA,
        ],
        compiler_params=pltpu.CompilerParams(
            use_tc_tiling_on_sc=False, needs_layout_passes=False
        ),
    )
    def k(idx_hbm, table_hbm, out_hbm, idx_v, rows0, rows1, tr0, tr1, gsem, ssem):
        wid = lax.axis_index("s") * NC + lax.axis_index("c")
        pltpu.sync_copy(idx_hbm.at[wid], idx_v)
        rows = (rows0, rows1)
        trs = (tr0, tr1)

        lane = lax.iota(jnp.int32, 16)

        def gather(j, p):
            return pltpu.make_async_copy(
                table_hbm.at[idx_v.at[j]], rows[p], gsem
            )

        def store(j, p):
            return pltpu.make_async_copy(
                trs[p], out_hbm.at[pl.ds(j * HC, HC), :, wid], ssem
            )

        def transpose(p):
            def body(i, carry):
                vb = lane + i * 16
                hq = i // (BT // 16)
                il = i % (BT // 16)
                xs = [
                    plsc.load_gather(
                        rows[p], [vb, jnp.full((16,), c, jnp.int32)]
                    )
                    for c in range(D)
                ]
                for c in range(D):
                    trs[p][hq, c // 8, c % 8, pl.ds(il * 16, 16)] = xs[c]
                return carry

            lax.fori_loop(0, RC // 16, body, 0)

        gather(0, 0).start()
        gather(1, 1).start()
        for p in (0, 1):
            gather(p, p).wait()
            transpose(p)
            store(p, p).start()
            gather(p + 2, p).start()

        def body(i, carry):
            for p in (0, 1):
                j = i * 2 + p
                gather(j, p).wait()
                store(j - 2, p).wait()
                transpose(p)
                store(j, p).start()
                gather(j + 2, p).start()
            return carry

        lax.fori_loop(1, nchunk // 2 - 1, body, 0)

        for p in (0, 1):
            j = nchunk - 2 + p
            gather(j, p).wait()
            store(j - 2, p).wait()
            transpose(p)
            store(j, p).start()
        store(nchunk - 2, 0).wait()
        store(nchunk - 1, 1).wait()

    return k


def kernel(words_pretrained, table):
    b0, hist = words_pretrained.shape
    nchunk = hist // HC
    idx_arr = (
        words_pretrained.T.astype(jnp.int32)
        .reshape(hist, NW, BT)
        .transpose(1, 0, 2)
        .reshape(NW, nchunk, RC)
    )
    vocab = table.shape[0]
    tlin = _retile_call(vocab)(table.T).reshape(vocab, D)
    out5 = _gather_call(hist, b0)(idx_arr, tlin)
    return out5.transpose(2, 4, 0, 1, 3).reshape(b0, hist, D)
